# Initial kernel scaffold; baseline (speedup 1.0000x reference)
import jax
import jax.numpy as jnp
from jax.experimental import pallas as pl


def kernel(x, edge_index, batch, W1, b1, W2, b2, W3, b3, W4, b4, W5, b5, W6, b6, g1, be1, g2, be2, g3, be3, g4, be4, g5, be5):
    raise NotImplementedError("write your pallas kernel here")


# speedup vs baseline: 8.3752x; 8.3752x over previous
import functools

import jax
import jax.numpy as jnp
from jax import lax
from jax.experimental import pallas as pl
from jax.experimental.pallas import tpu as pltpu
from jax.experimental.pallas import tpu_sc as plsc

N = 10000
D = 128
G = 64
EPS = 1e-5

NC = 2
NS = 16
NW = NC * NS
K = 128

ACC_ROWS = 10112
ZR = ACC_ROWS // NS

_sc_mesh = plsc.VectorSubcoreMesh(core_axis_name="c", subcore_axis_name="s")


def _deg_body(dst3, ones_rows, zrows, out, dslab, obuf, acc):
    c = lax.axis_index("c")
    s = lax.axis_index("s")
    w = s * NC + c
    ch = dst3.shape[1]
    pltpu.sync_copy(zrows.at[pl.ds(s * ZR, ZR)], acc.at[pl.ds(s * ZR, ZR)])
    pltpu.sync_copy(ones_rows, obuf)
    pltpu.sync_copy(dst3.at[w], dslab)
    plsc.subcore_barrier()

    def body(j, carry):
        pltpu.sync_copy(obuf, acc.at[dslab.at[j]], add=True)
        return carry

    lax.fori_loop(0, ch, body, 0)
    plsc.subcore_barrier()
    pltpu.sync_copy(acc.at[pl.ds(s * ZR, ZR)],
                    out.at[pl.ds(c * ACC_ROWS + s * ZR, ZR)])


def _scatter_body(hp, src3, dst3, zrows, out, sslab, dslab, rows, acc, sem):
    c = lax.axis_index("c")
    s = lax.axis_index("s")
    w = s * NC + c
    ch = src3.shape[1]
    pltpu.sync_copy(zrows.at[pl.ds(s * ZR, ZR)], acc.at[pl.ds(s * ZR, ZR)])
    pltpu.sync_copy(src3.at[w], sslab)
    pltpu.sync_copy(dst3.at[w], dslab)
    plsc.subcore_barrier()

    def body(j, carry):
        pltpu.async_copy(hp.at[sslab.at[j]], rows, sem).wait()
        pltpu.sync_copy(rows, acc.at[dslab.at[j]], add=True)
        return carry

    lax.fori_loop(0, ch, body, 0)
    plsc.subcore_barrier()
    pltpu.sync_copy(acc.at[pl.ds(s * ZR, ZR)],
                    out.at[pl.ds(c * ACC_ROWS + s * ZR, ZR)])


def _sc_degree(dst3, ones_rows, zrows):
    ch = dst3.shape[1]
    f = pl.kernel(
        _deg_body,
        out_type=jax.ShapeDtypeStruct((NC * ACC_ROWS, D), jnp.float32),
        mesh=_sc_mesh,
        scratch_types=[
            pltpu.VMEM((ch, K), jnp.int32),
            pltpu.VMEM((K, D), jnp.float32),
            pltpu.VMEM_SHARED((ACC_ROWS, D), jnp.float32),
        ],
    )
    return f(dst3, ones_rows, zrows)


def _sc_scatter(hp, src3, dst3, zrows):
    ch = src3.shape[1]
    f = pl.kernel(
        _scatter_body,
        out_type=jax.ShapeDtypeStruct((NC * ACC_ROWS, D), jnp.float32),
        mesh=_sc_mesh,
        scratch_types=[
            pltpu.VMEM((ch, K), jnp.int32),
            pltpu.VMEM((ch, K), jnp.int32),
            pltpu.VMEM((K, D), jnp.float32),
            pltpu.VMEM_SHARED((ACC_ROWS, D), jnp.float32),
            pltpu.SemaphoreType.DMA,
        ],
    )
    return f(hp, src3, dst3, zrows)


def _prep_body(x_ref, degp_ref, w_ref, dinv_ref, hp_ref):
    deg = 1.0 + degp_ref[0:N, 0:1] + degp_ref[ACC_ROWS:ACC_ROWS + N, 0:1]
    dinv = lax.rsqrt(deg)
    dinv_ref[...] = dinv
    h = jnp.dot(x_ref[...], w_ref[...], preferred_element_type=jnp.float32)
    hp_ref[...] = dinv * h


def _tc_prep(x, degp, w1):
    return pl.pallas_call(
        _prep_body,
        out_shape=[
            jax.ShapeDtypeStruct((N, 1), jnp.float32),
            jax.ShapeDtypeStruct((N, D), jnp.float32),
        ],
    )(x, degp, w1)


def _mid_body(sp_ref, hp_ref, dinv_ref, b_ref, g_ref, be_ref, w_ref, out_ref):
    dinv = dinv_ref[...]
    y = dinv * (sp_ref[0:N, :] + sp_ref[ACC_ROWS:ACC_ROWS + N, :]
                + hp_ref[...]) + b_ref[...]
    y = jnp.maximum(y, 0.0)
    m = jnp.mean(y, axis=0, keepdims=True)
    yc = y - m
    v = jnp.mean(yc * yc, axis=0, keepdims=True)
    z = yc * lax.rsqrt(v + EPS) * g_ref[...] + be_ref[...]
    out_ref[...] = dinv * jnp.dot(z, w_ref[...],
                                  preferred_element_type=jnp.float32)


def _tc_mid(sp, hp, dinv, b, g, be, w_next):
    return pl.pallas_call(
        _mid_body,
        out_shape=jax.ShapeDtypeStruct((N, D), jnp.float32),
    )(sp, hp, dinv, b, g, be, w_next)


def _final_body(sp_ref, hp_ref, dinv_ref, b_ref, batch_ref, out_ref):
    y = dinv_ref[...] * (sp_ref[0:N, :] + sp_ref[ACC_ROWS:ACC_ROWS + N, :]
                         + hp_ref[...]) + b_ref[...]
    y = jnp.maximum(y, 0.0)
    ids = lax.broadcasted_iota(jnp.int32, (N, G), 1)
    onehot = (batch_ref[...] == ids).astype(jnp.float32)
    out_ref[...] = lax.dot_general(
        onehot, y, (((0,), (0,)), ((), ())),
        preferred_element_type=jnp.float32)


def _tc_final(sp, hp, dinv, b, batch2d):
    return pl.pallas_call(
        _final_body,
        out_shape=jax.ShapeDtypeStruct((G, D), jnp.float32),
    )(sp, hp, dinv, b, batch2d)


def kernel(x, edge_index, batch, W1, b1, W2, b2, W3, b3, W4, b4, W5, b5,
           W6, b6, g1, be1, g2, be2, g3, be3, g4, be4, g5, be5):
    e = edge_index.shape[1]
    ch = -(-e // (NW * K))
    e_pad = NW * K * ch
    src = edge_index[0]
    dst = edge_index[1]
    pad = e_pad - e
    src3 = jnp.concatenate(
        [src, jnp.zeros((pad,), src.dtype)]).reshape(NW, ch, K)
    dst3 = jnp.concatenate(
        [dst, jnp.full((pad,), N, dst.dtype)]).reshape(NW, ch, K)

    ones_rows = jnp.ones((K, D), jnp.float32)
    zrows = jnp.zeros((ACC_ROWS, D), jnp.float32)

    degp = _sc_degree(dst3, ones_rows, zrows)
    dinv, hp = _tc_prep(x, degp, W1)

    ws = (W2, W3, W4, W5, W6)
    bs = (b1, b2, b3, b4, b5)
    gs = (g1, g2, g3, g4, g5)
    bes = (be1, be2, be3, be4, be5)
    for i in range(5):
        sp = _sc_scatter(hp, src3, dst3, zrows)
        hp = _tc_mid(sp, hp, dinv, bs[i], gs[i], bes[i], ws[i])
    sp = _sc_scatter(hp, src3, dst3, zrows)
    return _tc_final(sp, hp, dinv, b6, batch.reshape(N, 1))
